# Initial kernel scaffold; baseline (speedup 1.0000x reference)
import jax
import jax.numpy as jnp
from jax.experimental import pallas as pl


def kernel(x1, x2, lut):
    raise NotImplementedError("write your pallas kernel here")


# speedup vs baseline: 6.4280x; 6.4280x over previous
import functools
import math

import jax
import jax.numpy as jnp
from jax import lax
from jax.experimental import pallas as pl
from jax.experimental.pallas import tpu as pltpu
from jax.experimental.pallas import tpu_sc as plsc

_EMBED = 128
_LANES = 16
_NC, _NS = 2, 16
_NW = _NC * _NS
_K = 128


@functools.cache
def _build(n_idx, vocab):
    per_w = n_idx // _NW
    nchunks = per_w // _K
    scale = jnp.float32(math.sqrt(_EMBED))
    mesh = plsc.VectorSubcoreMesh(
        core_axis_name="c", subcore_axis_name="s",
        num_cores=_NC, num_subcores=_NS)

    @functools.partial(
        pl.kernel,
        out_type=jax.ShapeDtypeStruct((n_idx, _EMBED), jnp.float32),
        mesh=mesh,
        scratch_types=[
            pltpu.VMEM((_K,), jnp.int32),
            pltpu.VMEM((_K,), jnp.int32),
            pltpu.VMEM((_K, _EMBED), jnp.float32),
            pltpu.VMEM((_K, _EMBED), jnp.float32),
            pltpu.SemaphoreType.DMA,
            pltpu.SemaphoreType.DMA,
        ],
    )
    def emb_kernel(x1_hbm, x2_hbm, lut_hbm, out_hbm,
                   idx1_v, idx2_v, rows1, rows2, sem1, sem2):
        wid = lax.axis_index("s") * _NC + lax.axis_index("c")
        base = wid * per_w

        def chunk(j, _):
            off = base + j * _K
            pltpu.sync_copy(x1_hbm.at[pl.ds(off, _K)], idx1_v)
            pltpu.sync_copy(x2_hbm.at[pl.ds(off, _K)], idx2_v)
            c1 = pltpu.async_copy(lut_hbm.at[idx1_v], rows1, sem1)
            c2 = pltpu.async_copy(lut_hbm.at[idx2_v], rows2, sem2)
            c1.wait()
            c2.wait()

            @plsc.parallel_loop(0, _K, unroll=2)
            def _row(r):
                for c in range(_EMBED // _LANES):
                    sl = pl.ds(c * _LANES, _LANES)
                    rows2[r, sl] = (rows2[r, sl] - rows1[r, sl]) * scale

            pltpu.sync_copy(rows2, out_hbm.at[pl.ds(off, _K)])
            return 0

        lax.fori_loop(0, nchunks, chunk, 0)

    return emb_kernel


def kernel(x1, x2, lut):
    b, t = x1.shape
    n_idx = b * t
    x1f = x1.reshape(n_idx).astype(jnp.int32)
    x2f = x2.reshape(n_idx).astype(jnp.int32)
    out = _build(n_idx, lut.shape[0])(x1f, x2f, lut)
    return out.reshape(b, t, _EMBED)


# speedup vs baseline: 11.8473x; 1.8431x over previous
import functools
import math

import jax
import jax.numpy as jnp
from jax import lax
from jax.experimental import pallas as pl
from jax.experimental.pallas import tpu as pltpu
from jax.experimental.pallas import tpu_sc as plsc

_EMBED = 128
_LANES = 16
_NC, _NS = 2, 16
_NW = _NC * _NS
_K = 128


@functools.cache
def _build(n_idx, vocab):
    per_w = n_idx // _NW
    nchunks = per_w // _K
    half = nchunks // 2
    scale = jnp.float32(math.sqrt(_EMBED))
    mesh = plsc.VectorSubcoreMesh(
        core_axis_name="c", subcore_axis_name="s",
        num_cores=_NC, num_subcores=_NS)

    @functools.partial(
        pl.kernel,
        out_type=jax.ShapeDtypeStruct((n_idx, _EMBED), jnp.float32),
        mesh=mesh,
        scratch_types=[
            pltpu.VMEM((nchunks, _K), jnp.int32),
            pltpu.VMEM((nchunks, _K), jnp.int32),
            [pltpu.VMEM((_K, _EMBED), jnp.float32) for _ in range(2)],
            [pltpu.VMEM((_K, _EMBED), jnp.float32) for _ in range(2)],
            pltpu.SemaphoreType.DMA,
            [pltpu.SemaphoreType.DMA for _ in range(2)],
            [pltpu.SemaphoreType.DMA for _ in range(2)],
        ],
    )
    def emb_kernel(x1_hbm, x2_hbm, lut_hbm, out_hbm,
                   idx1_all, idx2_all, r1, r2, isem, gsem, osem):
        wid = lax.axis_index("s") * _NC + lax.axis_index("c")
        base = wid * per_w

        ic1 = pltpu.async_copy(
            x1_hbm.at[pl.ds(wid * nchunks, nchunks)], idx1_all, isem)
        ic2 = pltpu.async_copy(
            x2_hbm.at[pl.ds(wid * nchunks, nchunks)], idx2_all, isem)
        ic1.wait()
        ic2.wait()

        def gstart(j, b):
            pltpu.async_copy(lut_hbm.at[idx1_all.at[j]], r1[b], gsem[b])
            pltpu.async_copy(lut_hbm.at[idx2_all.at[j]], r2[b], gsem[b])

        def gwait(b):
            pltpu.make_async_copy(lut_hbm.at[pl.ds(0, _K)], r1[b], gsem[b]).wait()
            pltpu.make_async_copy(lut_hbm.at[pl.ds(0, _K)], r2[b], gsem[b]).wait()

        def ostart(j, b):
            pltpu.async_copy(r2[b], out_hbm.at[pl.ds(base + j * _K, _K)], osem[b])

        def owait(b):
            pltpu.make_async_copy(r2[b], out_hbm.at[pl.ds(0, _K)], osem[b]).wait()

        def compute(b):
            rows1, rows2 = r1[b], r2[b]

            @plsc.parallel_loop(0, _K, unroll=2)
            def _row(r):
                for c in range(_EMBED // _LANES):
                    sl = pl.ds(c * _LANES, _LANES)
                    rows2[r, sl] = (rows2[r, sl] - rows1[r, sl]) * scale

        gstart(0, 0)

        def step(jo, _):
            j0 = 2 * jo
            gwait(0)

            @pl.when(jo > 0)
            def _():
                owait(1)

            gstart(j0 + 1, 1)
            compute(0)
            ostart(j0, 0)
            gwait(1)

            @pl.when(jo < half - 1)
            def _():
                owait(0)
                gstart(j0 + 2, 0)

            compute(1)
            ostart(j0 + 1, 1)
            return 0

        lax.fori_loop(0, half, step, 0)
        owait(0)
        owait(1)

    return emb_kernel


def kernel(x1, x2, lut):
    b, t = x1.shape
    n_idx = b * t
    x1f = x1.reshape(n_idx // _K, _K).astype(jnp.int32)
    x2f = x2.reshape(n_idx // _K, _K).astype(jnp.int32)
    out = _build(n_idx, lut.shape[0])(x1f, x2f, lut)
    return out.reshape(b, t, _EMBED)


# speedup vs baseline: 15.3303x; 1.2940x over previous
import functools
import math

import jax
import jax.numpy as jnp
from jax import lax
from jax.experimental import pallas as pl
from jax.experimental.pallas import tpu as pltpu
from jax.experimental.pallas import tpu_sc as plsc

_EMBED = 128
_LANES = 16
_NC, _NS = 2, 16
_NW = _NC * _NS
_K = 128
_VPAD = 6272


@functools.cache
def _build(n_idx, vocab):
    per_w = n_idx // _NW
    nchunks = per_w // _K
    half = nchunks // 2
    stage_chunks = _VPAD // _K
    scale = jnp.float32(math.sqrt(_EMBED))
    mesh = plsc.VectorSubcoreMesh(
        core_axis_name="c", subcore_axis_name="s",
        num_cores=_NC, num_subcores=_NS)

    @functools.partial(
        pl.kernel,
        out_type=jax.ShapeDtypeStruct((n_idx, _EMBED), jnp.float32),
        mesh=mesh,
        scratch_types=[
            pltpu.VMEM_SHARED((_VPAD, _EMBED), jnp.float32),
            [pltpu.VMEM((_K,), jnp.int32) for _ in range(2)],
            [pltpu.VMEM((_K,), jnp.int32) for _ in range(2)],
            [pltpu.VMEM((_K, _EMBED), jnp.float32) for _ in range(2)],
            [pltpu.VMEM((_K, _EMBED), jnp.float32) for _ in range(2)],
            [pltpu.SemaphoreType.DMA for _ in range(2)],
            [pltpu.SemaphoreType.DMA for _ in range(2)],
            [pltpu.SemaphoreType.DMA for _ in range(2)],
        ],
    )
    def emb_kernel(x1_hbm, x2_hbm, lut_hbm, out_hbm, table_sh,
                   idx1, idx2, r1, r2, isem, gsem, osem):
        cid = lax.axis_index("c")
        sid = lax.axis_index("s")
        wid = sid * _NC + cid
        base = wid * per_w
        row0 = wid * nchunks

        def stage(s, _):
            c0 = s * _NS + sid

            @pl.when(c0 < stage_chunks)
            def _():
                off = c0 * _K
                pltpu.sync_copy(lut_hbm.at[pl.ds(off, _K)], r1[0])

                @plsc.parallel_loop(0, _K, unroll=2)
                def _row(r):
                    for c in range(_EMBED // _LANES):
                        sl = pl.ds(c * _LANES, _LANES)
                        r1[0][r, sl] = r1[0][r, sl] * scale

                pltpu.sync_copy(r1[0], table_sh.at[pl.ds(off, _K)])

            return 0

        lax.fori_loop(0, (stage_chunks + _NS - 1) // _NS, stage, 0)
        plsc.subcore_barrier()

        def istart(j, b):
            pltpu.async_copy(x1_hbm.at[row0 + j], idx1[b], isem[b])
            pltpu.async_copy(x2_hbm.at[row0 + j], idx2[b], isem[b])

        def iwait(b):
            pltpu.make_async_copy(x1_hbm.at[0], idx1[b], isem[b]).wait()
            pltpu.make_async_copy(x1_hbm.at[0], idx2[b], isem[b]).wait()

        def gstart(b):
            pltpu.async_copy(table_sh.at[idx1[b]], r1[b], gsem[b])
            pltpu.async_copy(table_sh.at[idx2[b]], r2[b], gsem[b])

        def gwait(b):
            pltpu.make_async_copy(table_sh.at[pl.ds(0, _K)], r1[b], gsem[b]).wait()
            pltpu.make_async_copy(table_sh.at[pl.ds(0, _K)], r2[b], gsem[b]).wait()

        def ostart(j, b):
            pltpu.async_copy(r2[b], out_hbm.at[pl.ds(base + j * _K, _K)], osem[b])

        def owait(b):
            pltpu.make_async_copy(r2[b], out_hbm.at[pl.ds(0, _K)], osem[b]).wait()

        def compute(b):
            rows1, rows2 = r1[b], r2[b]

            @plsc.parallel_loop(0, _K, unroll=2)
            def _row(r):
                for c in range(_EMBED // _LANES):
                    sl = pl.ds(c * _LANES, _LANES)
                    rows2[r, sl] = rows2[r, sl] - rows1[r, sl]

        istart(0, 0)
        iwait(0)
        gstart(0)
        istart(1, 1)

        def step(jo, _):
            j0 = 2 * jo
            gwait(0)

            @pl.when(jo < half - 1)
            def _():
                istart(j0 + 2, 0)

            @pl.when(jo > 0)
            def _():
                owait(1)

            iwait(1)
            gstart(1)
            compute(0)
            ostart(j0, 0)
            gwait(1)

            @pl.when(jo < half - 1)
            def _():
                istart(j0 + 3, 1)
                owait(0)
                iwait(0)
                gstart(0)

            compute(1)
            ostart(j0 + 1, 1)
            return 0

        lax.fori_loop(0, half, step, 0)
        owait(0)
        owait(1)

    return emb_kernel


def kernel(x1, x2, lut):
    b, t = x1.shape
    n_idx = b * t
    x1f = x1.reshape(n_idx // _K, _K).astype(jnp.int32)
    x2f = x2.reshape(n_idx // _K, _K).astype(jnp.int32)
    lut_pad = jnp.pad(lut, ((0, _VPAD - lut.shape[0]), (0, 0)))
    out = _build(n_idx, lut.shape[0])(x1f, x2f, lut_pad)
    return out.reshape(b, t, _EMBED)
